# baseline (speedup 1.0000x reference)
import functools

import jax
import jax.numpy as jnp
from jax import lax
from jax.experimental import pallas as pl
from jax.experimental.pallas import tpu as pltpu
from jax.experimental.pallas import tpu_sc as plsc

NC = 2
NS = 16
L = 16
NW = NC * NS

B = 16384
D = 64
BPW = B // NW
CH = 128
NCH = BPW // CH


def _decoder_body(z_hbm, h_hbm, t_hbm, out_hbm, idx_v, zh_v, zt_v, out_v, sem):
    wid = lax.axis_index("s") * NC + lax.axis_index("c")
    base = wid * BPW

    pltpu.sync_copy(h_hbm.at[pl.ds(wid * NCH, NCH)], idx_v.at[pl.ds(0, NCH)])
    pltpu.sync_copy(t_hbm.at[pl.ds(wid * NCH, NCH)], idx_v.at[pl.ds(NCH, NCH)])

    copies = []
    for j in range(NCH):
        copies.append(
            pltpu.async_copy(z_hbm.at[idx_v.at[j]],
                             zh_v.at[pl.ds(j * CH, CH)], sem))
    for j in range(NCH):
        copies.append(
            pltpu.async_copy(z_hbm.at[idx_v.at[NCH + j]],
                             zt_v.at[pl.ds(j * CH, CH)], sem))
    for c in copies:
        c.wait()

    lanes = lax.iota(jnp.int32, L)

    def group(g, carry):
        res = jnp.zeros((L,), jnp.float32)
        for j in range(L):
            i = g * L + j
            acc = jnp.zeros((L,), jnp.float32)
            for c in range(D // L):
                acc = acc + (zh_v[i, pl.ds(c * L, L)]
                             * zt_v[i, pl.ds(c * L, L)])
            res = jnp.where(lanes == j, jnp.sum(acc), res)
        out_v[pl.ds(g * L, L)] = res
        return carry

    lax.fori_loop(0, BPW // L, group, 0)

    pltpu.sync_copy(out_v, out_hbm.at[pl.ds(base, BPW)])


@functools.partial(jax.jit, static_argnames=())
def _decode(z, h, t):
    mesh = plsc.VectorSubcoreMesh(core_axis_name="c", subcore_axis_name="s",
                                  num_cores=NC, num_subcores=NS)
    return pl.kernel(
        _decoder_body,
        out_type=jax.ShapeDtypeStruct((B,), jnp.float32),
        mesh=mesh,
        compiler_params=pltpu.CompilerParams(needs_layout_passes=False,
                                             use_tc_tiling_on_sc=False),
        scratch_types=[
            pltpu.VMEM((2 * NCH, CH), jnp.int32),
            pltpu.VMEM((BPW, D), jnp.float32),
            pltpu.VMEM((BPW, D), jnp.float32),
            pltpu.VMEM((BPW,), jnp.float32),
            pltpu.SemaphoreType.DMA,
        ],
    )(z, h, t)


def kernel(z, triples):
    h = triples[:, 0].astype(jnp.int32).reshape(NW * NCH, CH)
    t = triples[:, 2].astype(jnp.int32).reshape(NW * NCH, CH)
    return _decode(z, h, t)
